# baseline (speedup 1.0000x reference)
import functools

import jax
import jax.numpy as jnp
from jax import lax
from jax.experimental import pallas as pl
from jax.experimental.pallas import tpu as pltpu
from jax.experimental.pallas import tpu_sc as plsc


@functools.lru_cache(maxsize=None)
def _make_sc_gather(V, D, B):
    info = plsc.get_sparse_core_info()
    NC, NS = 1, info.num_subcores
    NW = NC * NS
    assert B % (8 * NW) == 0
    b_per_w = B // NW
    mesh = plsc.VectorSubcoreMesh(
        core_axis_name="c", subcore_axis_name="s", num_cores=1
    )

    @functools.partial(
        pl.kernel,
        mesh=mesh,
        out_type=jax.ShapeDtypeStruct((B, D), jnp.float32),
        scratch_types=[
            pltpu.VMEM((b_per_w,), jnp.int32),
            pltpu.VMEM((b_per_w, D), jnp.float32),
            pltpu.SemaphoreType.DMA,
        ],
    )
    def gather(table_hbm, idx_hbm, out_hbm, idx_v, rows_v, sem):
        wid = lax.axis_index("s") * NC + lax.axis_index("c")
        base = wid * b_per_w
        pltpu.sync_copy(idx_hbm.at[pl.ds(base, b_per_w)], idx_v)
        pltpu.async_copy(table_hbm.at[idx_v], rows_v, sem).wait()
        pltpu.sync_copy(rows_v, out_hbm.at[pl.ds(base, b_per_w)])

    return gather


_BIG = 3.0e38


def _dot(a, b, prec):
    return lax.dot_general(
        a, b, (((1,), (1,)), ((), ())),
        preferred_element_type=jnp.float32, precision=prec,
    )


def _dist_body(x_ref, c_ref, dist_ref, ids_ref):
    for i in range(x_ref.shape[0]):
        _dist_one(x_ref[i], c_ref[i], dist_ref.at[i], ids_ref.at[i])


def _dist_one(x, c, dist_ref, ids_ref):
    N, F = x.shape
    K = c.shape[0]
    hi = lax.Precision.HIGHEST
    x2 = jnp.sum(x * x, axis=1, keepdims=True)
    c2 = jnp.sum(c * c, axis=1)[None, :]
    g = _dot(x, c, hi)
    d2 = jnp.maximum(x2 + c2 - 2.0 * g, 0.0)
    dist = jnp.sqrt(d2)
    dist_ref[...] = dist
    kf = lax.broadcasted_iota(jnp.int32, (N, K), 1).astype(jnp.float32)
    fK = float(K)
    m1 = jnp.min(dist, axis=1, keepdims=True)
    k1 = jnp.min(jnp.where(dist == m1, kf, fK), axis=1, keepdims=True)
    mask1 = kf == k1
    dist_x = jnp.where(mask1, _BIG, dist)
    m2 = jnp.min(dist_x, axis=1, keepdims=True)
    k2 = jnp.min(jnp.where(dist_x == m2, kf, fK), axis=1, keepdims=True)
    mask2 = kf == k2
    c0 = c.astype(jnp.bfloat16)
    r1 = c - c0.astype(jnp.float32)
    c1 = r1.astype(jnp.bfloat16)
    c2b = (r1 - c1.astype(jnp.float32)).astype(jnp.bfloat16)

    def gath(mask):
        oh = mask.astype(jnp.float32).astype(jnp.bfloat16)
        acc = lax.dot_general(
            oh, c0, (((1,), (0,)), ((), ())),
            preferred_element_type=jnp.float32)
        for cc in (c1, c2b):
            acc = acc + lax.dot_general(
                oh, cc, (((1,), (0,)), ((), ())),
                preferred_element_type=jnp.float32)
        return acc

    ids_ref[...] = k1.astype(jnp.int32)
    gap = m2 - m1
    CH = 256
    for j in range(N // CH):
        lo, hi_ = j * CH, (j + 1) * CH

        @pl.when(jnp.min(gap[lo:hi_]) < 1e-4)
        def _refine(lo=lo, hi_=hi_):
            xc = x[lo:hi_]
            z1 = xc - gath(mask1[lo:hi_])
            z2 = xc - gath(mask2[lo:hi_])
            s1 = jnp.sqrt(jnp.sum(z1 * z1, axis=1, keepdims=True))
            s2 = jnp.sqrt(jnp.sum(z2 * z2, axis=1, keepdims=True))
            k1c, k2c = k1[lo:hi_], k2[lo:hi_]
            ids = jnp.where(s2 < s1, k2c, k1c)
            ids = jnp.where(s1 == s2, jnp.minimum(k1c, k2c), ids)
            ids_ref[lo:hi_] = ids.astype(jnp.int32)


def _distance(data, cents):
    B, N, F = data.shape
    K = cents.shape[1]
    return pl.pallas_call(
        _dist_body,
        grid=(B,),
        in_specs=[
            pl.BlockSpec((1, N, F), lambda b: (b, 0, 0)),
            pl.BlockSpec((1, K, F), lambda b: (b, 0, 0)),
        ],
        out_specs=[
            pl.BlockSpec((1, N, K), lambda b: (b, 0, 0)),
            pl.BlockSpec((1, N, 1), lambda b: (b, 0, 0)),
        ],
        out_shape=[
            jax.ShapeDtypeStruct((B, N, K), jnp.float32),
            jax.ShapeDtypeStruct((B, N, 1), jnp.int32),
        ],
    )(data, cents)


def kernel(data, centroid_ids):
    B, N, F = data.shape
    K = centroid_ids.shape[1]
    flat_ids = centroid_ids.reshape(B * K)
    table = data.reshape(B * N, F)
    cents = _make_sc_gather(B * N, F, B * K)(table, flat_ids)
    dist, ids3 = _distance(data, cents.reshape(B, K, F))
    return dist, ids3.reshape(B, N)


# speedup vs baseline: 1.1299x; 1.1299x over previous
import functools

import jax
import jax.numpy as jnp
from jax import lax
from jax.experimental import pallas as pl
from jax.experimental.pallas import tpu as pltpu
from jax.experimental.pallas import tpu_sc as plsc


@functools.lru_cache(maxsize=None)
def _make_sc_gather(V, D, B):
    info = plsc.get_sparse_core_info()
    NC, NS = 1, info.num_subcores
    NW = NC * NS
    assert B % (8 * NW) == 0
    b_per_w = B // NW
    mesh = plsc.VectorSubcoreMesh(
        core_axis_name="c", subcore_axis_name="s", num_cores=1
    )

    @functools.partial(
        pl.kernel,
        mesh=mesh,
        out_type=jax.ShapeDtypeStruct((B, D), jnp.float32),
        scratch_types=[
            pltpu.VMEM((b_per_w,), jnp.int32),
            pltpu.VMEM((b_per_w, D), jnp.float32),
            pltpu.SemaphoreType.DMA,
        ],
    )
    def gather(table_hbm, idx_hbm, out_hbm, idx_v, rows_v, sem):
        wid = lax.axis_index("s") * NC + lax.axis_index("c")
        base = wid * b_per_w
        pltpu.sync_copy(idx_hbm.at[pl.ds(base, b_per_w)], idx_v)
        pltpu.async_copy(table_hbm.at[idx_v], rows_v, sem).wait()
        pltpu.sync_copy(rows_v, out_hbm.at[pl.ds(base, b_per_w)])

    return gather


_BIG = 3.0e38


def _dot(a, b, prec):
    return lax.dot_general(
        a, b, (((1,), (1,)), ((), ())),
        preferred_element_type=jnp.float32, precision=prec,
    )


def _dist_body(x_ref, c_ref, dup_ref, dist_ref, ids_ref):
    for i in range(x_ref.shape[0]):
        _dist_one(x_ref[i], c_ref[i], dup_ref[i], dist_ref.at[i], ids_ref.at[i])


def _dist_one(x, c, dup, dist_ref, ids_ref):
    N, F = x.shape
    K = c.shape[0]
    hi = lax.Precision.HIGHEST
    x2 = jnp.sum(x * x, axis=1, keepdims=True)
    c2 = jnp.sum(c * c, axis=1)[None, :]
    g = _dot(x, c, hi)
    d2 = jnp.maximum(x2 + c2 - 2.0 * g, 0.0)
    dist = jnp.sqrt(d2)
    dist_ref[...] = dist
    kf = lax.broadcasted_iota(jnp.int32, (N, K), 1).astype(jnp.float32)
    fK = float(K)
    dist_a = jnp.where(dup == 1.0, _BIG, dist)
    m1 = jnp.min(dist_a, axis=1, keepdims=True)
    k1 = jnp.min(jnp.where(dist_a == m1, kf, fK), axis=1, keepdims=True)
    mask1 = kf == k1
    dist_x = jnp.where(mask1, _BIG, dist_a)
    m2 = jnp.min(dist_x, axis=1, keepdims=True)
    k2 = jnp.min(jnp.where(dist_x == m2, kf, fK), axis=1, keepdims=True)
    mask2 = kf == k2
    c0 = c.astype(jnp.bfloat16)
    r1 = c - c0.astype(jnp.float32)
    c1 = r1.astype(jnp.bfloat16)
    c2b = (r1 - c1.astype(jnp.float32)).astype(jnp.bfloat16)

    def gath(mask):
        oh = mask.astype(jnp.float32).astype(jnp.bfloat16)
        acc = lax.dot_general(
            oh, c0, (((1,), (0,)), ((), ())),
            preferred_element_type=jnp.float32)
        for cc in (c1, c2b):
            acc = acc + lax.dot_general(
                oh, cc, (((1,), (0,)), ((), ())),
                preferred_element_type=jnp.float32)
        return acc

    ids_ref[...] = k1.astype(jnp.int32)
    gap = m2 - m1
    CH = 256
    for j in range(N // CH):
        lo, hi_ = j * CH, (j + 1) * CH

        @pl.when(jnp.min(gap[lo:hi_]) < 1e-4)
        def _refine(lo=lo, hi_=hi_):
            xc = x[lo:hi_]
            z1 = xc - gath(mask1[lo:hi_])
            z2 = xc - gath(mask2[lo:hi_])
            s1 = jnp.sqrt(jnp.sum(z1 * z1, axis=1, keepdims=True))
            s2 = jnp.sqrt(jnp.sum(z2 * z2, axis=1, keepdims=True))
            k1c, k2c = k1[lo:hi_], k2[lo:hi_]
            ids = jnp.where(s2 < s1, k2c, k1c)
            ids = jnp.where(s1 == s2, jnp.minimum(k1c, k2c), ids)
            ids_ref[lo:hi_] = ids.astype(jnp.int32)


def _distance(data, cents, dup):
    B, N, F = data.shape
    K = cents.shape[1]
    return pl.pallas_call(
        _dist_body,
        grid=(B,),
        in_specs=[
            pl.BlockSpec((1, N, F), lambda b: (b, 0, 0)),
            pl.BlockSpec((1, K, F), lambda b: (b, 0, 0)),
            pl.BlockSpec((1, 1, K), lambda b: (b, 0, 0)),
        ],
        out_specs=[
            pl.BlockSpec((1, N, K), lambda b: (b, 0, 0)),
            pl.BlockSpec((1, N, 1), lambda b: (b, 0, 0)),
        ],
        out_shape=[
            jax.ShapeDtypeStruct((B, N, K), jnp.float32),
            jax.ShapeDtypeStruct((B, N, 1), jnp.int32),
        ],
    )(data, cents, dup)


def kernel(data, centroid_ids):
    B, N, F = data.shape
    K = centroid_ids.shape[1]
    flat_ids = centroid_ids.reshape(B * K)
    table = data.reshape(B * N, F)
    cents = _make_sc_gather(B * N, F, B * K)(table, flat_ids)
    kk = jnp.arange(K)
    eq = centroid_ids[:, :, None] == centroid_ids[:, None, :]
    dup = jnp.any(eq & (kk[None, :] < kk[:, None])[None], axis=2)
    dup = dup.astype(jnp.float32).reshape(B, 1, K)
    dist, ids3 = _distance(data, cents.reshape(B, K, F), dup)
    return dist, ids3.reshape(B, N)
